# Initial kernel scaffold; baseline (speedup 1.0000x reference)
import jax
import jax.numpy as jnp
from jax.experimental import pallas as pl


def kernel(x, indices):
    raise NotImplementedError("write your pallas kernel here")


# speedup vs baseline: 1.6511x; 1.6511x over previous
import functools

import jax
import jax.numpy as jnp
from jax import lax
from jax.experimental import pallas as pl
from jax.experimental.pallas import tpu as pltpu
from jax.experimental.pallas import tpu_sc as plsc

NC, NS, L = 2, 16, 16
NW = NC * NS
BATCH, INPUTS, OUTPUTS = 1024, 16384, 32768
ROWS_PER_W = BATCH // NW
HALF = OUTPUTS // 2


def _gather_body(x_hbm, idx_hbm, out_hbm, idx_v, row_v, out_v):
    wid = lax.axis_index("s") * NC + lax.axis_index("c")
    pltpu.sync_copy(idx_hbm, idx_v)

    def do_row(r, carry):
        b = wid * ROWS_PER_W + r
        pltpu.sync_copy(x_hbm.at[b], row_v)

        def do_half(h, carry2):
            @plsc.parallel_loop(0, HALF, step=L, unroll=8)
            def _chunk(j):
                idx = idx_v[pl.ds(h * HALF + j, L)]
                out_v[pl.ds(j, L)] = plsc.load_gather(row_v, [idx])

            pltpu.sync_copy(out_v, out_hbm.at[b, pl.ds(h * HALF, HALF)])
            return carry2

        lax.fori_loop(0, 2, do_half, 0)
        return carry

    lax.fori_loop(0, ROWS_PER_W, do_row, 0)


_gather_call = functools.partial(
    pl.kernel,
    out_type=jax.ShapeDtypeStruct((BATCH, OUTPUTS), jnp.float32),
    mesh=plsc.VectorSubcoreMesh(
        core_axis_name="c", subcore_axis_name="s",
        num_cores=NC, num_subcores=NS,
    ),
    scratch_types=[
        pltpu.VMEM((OUTPUTS,), jnp.int32),
        pltpu.VMEM((INPUTS,), jnp.float32),
        pltpu.VMEM((HALF,), jnp.float32),
    ],
    compiler_params=pltpu.CompilerParams(needs_layout_passes=False),
)(_gather_body)


def kernel(x, indices):
    return _gather_call(x, indices)


# speedup vs baseline: 2.6639x; 1.6135x over previous
import functools

import jax
import jax.numpy as jnp
from jax import lax
from jax.experimental import pallas as pl
from jax.experimental.pallas import tpu as pltpu
from jax.experimental.pallas import tpu_sc as plsc

NC, NS, L = 2, 16, 16
NW = NC * NS
BATCH, INPUTS, OUTPUTS = 1024, 16384, 32768
ROWS_PER_W = BATCH // NW
HALF = OUTPUTS // 2


def _gather_body(x_hbm, idx_hbm, out_hbm, idx_v, row0_v, row1_v,
                 outa_v, outb_v, in_sem0, in_sem1, out_sem0, out_sem1):
    wid = lax.axis_index("s") * NC + lax.axis_index("c")
    base = wid * ROWS_PER_W
    pltpu.sync_copy(idx_hbm, idx_v)

    rows = (row0_v, row1_v)
    outs = (outa_v, outb_v)
    in_sems = (in_sem0, in_sem1)
    out_sems = (out_sem0, out_sem1)
    in_copies = [None, None]
    out_copies = [None, None]

    in_copies[0] = pltpu.async_copy(x_hbm.at[base], rows[0], in_sems[0])
    for r in range(ROWS_PER_W):
        cur = r & 1
        if r + 1 < ROWS_PER_W:
            in_copies[1 - cur] = pltpu.async_copy(
                x_hbm.at[base + r + 1], rows[1 - cur], in_sems[1 - cur])
        in_copies[cur].wait()
        for h in range(2):
            if out_copies[h] is not None:
                out_copies[h].wait()
            row_ref = rows[cur]
            out_ref = outs[h]

            @plsc.parallel_loop(0, HALF, step=L, unroll=8)
            def _chunk(j, h=h, row_ref=row_ref, out_ref=out_ref):
                idx = idx_v[pl.ds(h * HALF + j, L)]
                out_ref[pl.ds(j, L)] = plsc.load_gather(row_ref, [idx])

            out_copies[h] = pltpu.async_copy(
                out_ref, out_hbm.at[base + r, pl.ds(h * HALF, HALF)],
                out_sems[h])
    for h in range(2):
        out_copies[h].wait()


_gather_call = functools.partial(
    pl.kernel,
    out_type=jax.ShapeDtypeStruct((BATCH, OUTPUTS), jnp.float32),
    mesh=plsc.VectorSubcoreMesh(
        core_axis_name="c", subcore_axis_name="s",
        num_cores=NC, num_subcores=NS,
    ),
    scratch_types=[
        pltpu.VMEM((OUTPUTS,), jnp.int32),
        pltpu.VMEM((INPUTS,), jnp.float32),
        pltpu.VMEM((INPUTS,), jnp.float32),
        pltpu.VMEM((HALF,), jnp.float32),
        pltpu.VMEM((HALF,), jnp.float32),
        pltpu.SemaphoreType.DMA,
        pltpu.SemaphoreType.DMA,
        pltpu.SemaphoreType.DMA,
        pltpu.SemaphoreType.DMA,
    ],
    compiler_params=pltpu.CompilerParams(needs_layout_passes=False),
)(_gather_body)


def kernel(x, indices):
    return _gather_call(x, indices)
